# baseline (speedup 1.0000x reference)
import jax
import jax.numpy as jnp
import numpy as np
from jax import lax
from jax.experimental import pallas as pl
from jax.experimental.pallas import tpu as pltpu
from jax.experimental.pallas import tpu_sc as plsc

VOCAB_ = 1000000
EMBED_ = 64
BATCH_ = 4096
SEQ_ = 200

NC = 2
NS = 16
LANES = 16
NW = NC * NS

CB = 256
CHUNKS_PER_S = BATCH_ // CB
N_TASKS = SEQ_ * CHUNKS_PER_S
TASKS_PW = N_TASKS // NW
NBUF = 2
PE_SPAN = TASKS_PW // CHUNKS_PER_S + 1
assert TASKS_PW % NBUF == 0


def _pos_encoding():
    position = np.arange(SEQ_, dtype=np.float32)[:, None]
    div_term = np.exp(
        np.arange(0, EMBED_, 2, dtype=np.float32) * (-np.log(10000.0) / EMBED_)
    )
    pe = np.zeros((SEQ_, EMBED_), dtype=np.float32)
    pe[:, 0::2] = np.sin(position * div_term)
    pe[:, 1::2] = np.cos(position * div_term)
    return jnp.asarray(pe)


def _sc_body(xf_hbm, pe_hbm, table_hbm, out_hbm,
             idx_all, rows0, rows1, tr0, tr1, pe_v, gsem, osem):
    rows = [rows0, rows1]
    tr = [tr0, tr1]
    wid = lax.axis_index("s") * NC + lax.axis_index("c")
    t_base = wid * TASKS_PW
    s_min = t_base // CHUNKS_PER_S

    pltpu.sync_copy(xf_hbm.at[pl.ds(t_base * CB, TASKS_PW * CB)], idx_all)
    pltpu.sync_copy(pe_hbm.at[pl.ds(s_min, PE_SPAN)], pe_v)

    def task_coords(t):
        tt = t_base + t
        s = tt // CHUNKS_PER_S
        b0 = (tt % CHUNKS_PER_S) * CB
        return s, b0

    def start_gather(b, t):
        pltpu.async_copy(
            table_hbm.at[idx_all.at[pl.ds(t * CB, CB)]], rows[b], gsem.at[b]
        )

    for b in range(NBUF):
        start_gather(b, b)

    e_idx = [lax.iota(jnp.int32, LANES) + q * LANES for q in range(EMBED_ // LANES)]

    @pl.loop(0, TASKS_PW, step=NBUF)
    def _ring(t0):
        for b in range(NBUF):
            t = t0 + b
            s, b0 = task_coords(t)
            pltpu.make_async_copy(
                table_hbm.at[idx_all.at[pl.ds(t * CB, CB)]], rows[b], gsem.at[b]
            ).wait()

            @pl.when(t >= NBUF)
            def _wb_done():
                sp, bp = task_coords(t - NBUF)
                pltpu.make_async_copy(
                    tr[b], out_hbm.at[sp, :, pl.ds(bp, CB)], osem.at[b]
                ).wait()

            pes = [pe_v[s - s_min, pl.ds(q * LANES, LANES)]
                   for q in range(EMBED_ // LANES)]

            @pl.loop(0, CB, unroll=4)
            def _row(r):
                bcol = jnp.full((LANES,), r, dtype=jnp.int32)
                for q in range(EMBED_ // LANES):
                    vals = rows[b][r, pl.ds(q * LANES, LANES)] + pes[q]
                    plsc.store_scatter(tr[b], [e_idx[q], bcol], vals)

            pltpu.async_copy(tr[b], out_hbm.at[s, :, pl.ds(b0, CB)], osem.at[b])

            @pl.when(t + NBUF < TASKS_PW)
            def _refill():
                start_gather(b, t + NBUF)

    for t in range(TASKS_PW - NBUF, TASKS_PW):
        b = t % NBUF
        s, b0 = task_coords(t)
        pltpu.make_async_copy(
            tr[b], out_hbm.at[s, :, pl.ds(b0, CB)], osem.at[b]
        ).wait()


@jax.jit
def _embed(x, table, pe):
    xf = jnp.transpose(x.astype(jnp.int32), (1, 0)).reshape(SEQ_ * BATCH_)
    mesh = plsc.VectorSubcoreMesh(core_axis_name="c", subcore_axis_name="s")
    out = pl.kernel(
        _sc_body,
        out_type=jax.ShapeDtypeStruct((SEQ_, EMBED_, BATCH_), jnp.float32),
        mesh=mesh,
        scratch_types=[
            pltpu.VMEM((TASKS_PW * CB,), jnp.int32),
            pltpu.VMEM((CB, EMBED_), jnp.float32),
            pltpu.VMEM((CB, EMBED_), jnp.float32),
            pltpu.VMEM((EMBED_, CB), jnp.float32),
            pltpu.VMEM((EMBED_, CB), jnp.float32),
            pltpu.VMEM((PE_SPAN, EMBED_), jnp.float32),
            pltpu.SemaphoreType.DMA((NBUF,)),
            pltpu.SemaphoreType.DMA((NBUF,)),
        ],
        compiler_params=pltpu.CompilerParams(
            use_tc_tiling_on_sc=False, needs_layout_passes=False
        ),
    )(xf, pe, table)
    return jnp.transpose(out, (2, 0, 1))


def kernel(x, table):
    return _embed(x, table, _pos_encoding())


# speedup vs baseline: 1.4562x; 1.4562x over previous
import jax
import jax.numpy as jnp
import numpy as np
from jax import lax
from jax.experimental import pallas as pl
from jax.experimental.pallas import tpu as pltpu
from jax.experimental.pallas import tpu_sc as plsc

VOCAB_ = 1000000
EMBED_ = 64
BATCH_ = 4096
SEQ_ = 200

NC = 2
NS = 16
LANES = 16
NW = NC * NS

CB = 256
CHUNKS_PER_S = BATCH_ // CB
N_TASKS = SEQ_ * CHUNKS_PER_S
TASKS_PW = N_TASKS // NW
NBUF = 2
PE_SPAN = TASKS_PW // CHUNKS_PER_S + 1
assert TASKS_PW % NBUF == 0


def _pos_encoding():
    position = np.arange(SEQ_, dtype=np.float32)[:, None]
    div_term = np.exp(
        np.arange(0, EMBED_, 2, dtype=np.float32) * (-np.log(10000.0) / EMBED_)
    )
    pe = np.zeros((SEQ_, EMBED_), dtype=np.float32)
    pe[:, 0::2] = np.sin(position * div_term)
    pe[:, 1::2] = np.cos(position * div_term)
    return jnp.asarray(pe)


def _sc_body(xf_hbm, pe_hbm, table_hbm, out_hbm,
             idx_all, rows0, rows1, tr0, tr1, pe_v, gsem, osem):
    rows = [rows0, rows1]
    tr = [tr0, tr1]
    wid = lax.axis_index("s") * NC + lax.axis_index("c")
    t_base = wid * TASKS_PW
    s_min = t_base // CHUNKS_PER_S

    pltpu.sync_copy(xf_hbm.at[pl.ds(t_base * CB, TASKS_PW * CB)], idx_all)
    pltpu.sync_copy(pe_hbm.at[pl.ds(s_min, PE_SPAN)], pe_v)

    def task_coords(t):
        tt = t_base + t
        s = tt // CHUNKS_PER_S
        b0 = (tt % CHUNKS_PER_S) * CB
        return s, b0

    def start_gather(b, t):
        pltpu.async_copy(
            table_hbm.at[idx_all.at[pl.ds(t * CB, CB)]], rows[b], gsem.at[b]
        )

    for b in range(NBUF):
        start_gather(b, b)

    e_idx = [lax.iota(jnp.int32, LANES) + q * LANES for q in range(EMBED_ // LANES)]

    @pl.loop(0, TASKS_PW, step=NBUF)
    def _ring(t0):
        for b in range(NBUF):
            t = t0 + b
            s, b0 = task_coords(t)
            pltpu.make_async_copy(
                table_hbm.at[idx_all.at[pl.ds(t * CB, CB)]], rows[b], gsem.at[b]
            ).wait()

            @pl.when(t >= NBUF)
            def _wb_done():
                sp, bp = task_coords(t - NBUF)
                pltpu.make_async_copy(
                    tr[b].at[:, pl.ds(0, CB)], out_hbm.at[sp, :, pl.ds(bp, CB)], osem.at[b]
                ).wait()

            pes = [pe_v[s - s_min, pl.ds(q * LANES, LANES)]
                   for q in range(EMBED_ // LANES)]

            @pl.loop(0, CB, unroll=4)
            def _row(r):
                bcol = jnp.full((LANES,), r, dtype=jnp.int32)
                for q in range(EMBED_ // LANES):
                    vals = rows[b][r, pl.ds(q * LANES, LANES)] + pes[q]
                    plsc.store_scatter(tr[b], [e_idx[q], bcol], vals)

            pltpu.async_copy(tr[b].at[:, pl.ds(0, CB)], out_hbm.at[s, :, pl.ds(b0, CB)], osem.at[b])

            @pl.when(t + NBUF < TASKS_PW)
            def _refill():
                start_gather(b, t + NBUF)

    for t in range(TASKS_PW - NBUF, TASKS_PW):
        b = t % NBUF
        s, b0 = task_coords(t)
        pltpu.make_async_copy(
            tr[b].at[:, pl.ds(0, CB)], out_hbm.at[s, :, pl.ds(b0, CB)], osem.at[b]
        ).wait()


@jax.jit
def _embed(x, table, pe):
    xf = jnp.transpose(x.astype(jnp.int32), (1, 0)).reshape(SEQ_ * BATCH_)
    mesh = plsc.VectorSubcoreMesh(core_axis_name="c", subcore_axis_name="s")
    out = pl.kernel(
        _sc_body,
        out_type=jax.ShapeDtypeStruct((SEQ_, EMBED_, BATCH_), jnp.float32),
        mesh=mesh,
        scratch_types=[
            pltpu.VMEM((TASKS_PW * CB,), jnp.int32),
            pltpu.VMEM((CB, EMBED_), jnp.float32),
            pltpu.VMEM((CB, EMBED_), jnp.float32),
            pltpu.VMEM((EMBED_, CB + 1), jnp.float32),
            pltpu.VMEM((EMBED_, CB + 1), jnp.float32),
            pltpu.VMEM((PE_SPAN, EMBED_), jnp.float32),
            pltpu.SemaphoreType.DMA((NBUF,)),
            pltpu.SemaphoreType.DMA((NBUF,)),
        ],
        compiler_params=pltpu.CompilerParams(
            use_tc_tiling_on_sc=False, needs_layout_passes=False
        ),
    )(xf, pe, table)
    return jnp.transpose(out, (2, 0, 1))


def kernel(x, table):
    return _embed(x, table, _pos_encoding())
